# baseline (speedup 1.0000x reference)
import functools

import jax
import jax.numpy as jnp
from jax import lax
from jax.experimental import pallas as pl
from jax.experimental.pallas import tpu as pltpu
from jax.experimental.pallas import tpu_sc as plsc

_NC = 2
_NS = 16
_NW = _NC * _NS
_LANES = 16
_BBLK = 128


_BURST = 8


def _body(L, B, nr_scale,
          table_hbm, idx_hbm, scale_hbm, shift_hbm, out_hbm,
          idx_v, rows_v, rowsT_v, sc_v, sh_v, sg):
    wid = lax.axis_index("s") * _NC + lax.axis_index("c")
    b0 = pl.multiple_of(wid * _BBLK, _BBLK)
    pltpu.sync_copy(scale_hbm, sc_v)
    pltpu.sync_copy(shift_hbm, sh_v)
    a0 = sc_v[0:_LANES]
    a1 = sc_v[_LANES:2 * _LANES]
    c0 = sh_v[0:_LANES]
    c1 = sh_v[_LANES:2 * _LANES]
    pltpu.sync_copy(idx_hbm.at[:, pl.ds(b0, _BBLK)], idx_v)

    lane = lax.iota(jnp.int32, _LANES)
    t0 = lax.shift_right_logical(lane, 3)
    t1 = t0 + 2
    r8 = lane & 7
    zero = jnp.zeros((_LANES,), jnp.int32)

    @pl.loop(0, L // _BURST)
    def _burst(t):
        l0 = t * _BURST
        copies = []
        for j in range(_BURST):
            copies.append(pltpu.async_copy(
                table_hbm.at[idx_v.at[l0 + j]],
                rows_v.at[pl.ds(j * _BBLK, _BBLK)], sg))
        for c in copies:
            c.wait()

        @pl.loop(0, _BURST * _BBLK // _LANES)
        def _grp(gg):
            lg = lax.shift_right_logical(gg, 3)
            within = (gg & 7) * _LANES
            ivec = idx_v[l0 + lg, pl.ds(within, _LANES)]
            nrv = ivec.astype(jnp.float32) * nr_scale - 1.0
            lgv = zero + lg
            for k in range(_LANES):
                nr = nrv[k]
                fr = gg * _LANES + k
                cb = zero + (within + k)
                v0 = rows_v[fr, 0:_LANES]
                v1 = rows_v[fr, _LANES:2 * _LANES]
                plsc.store_scatter(rowsT_v, [lgv, t0, zero, r8, cb],
                                   v0 + (nr * a0 + c0))
                plsc.store_scatter(rowsT_v, [lgv, t1, zero, r8, cb],
                                   v1 + (nr * a1 + c1))

        pltpu.sync_copy(rowsT_v,
                        out_hbm.at[pl.ds(l0, _BURST), :, pl.ds(wid, 1)])


def kernel(class_embedding, order_embedding, bn_weight, bn_bias, index_tensor):
    V, D = class_embedding.shape
    B, L = index_tensor.shape
    assert B == _NW * _BBLK and D == 2 * _LANES and L % _BURST == 0

    r = jax.nn.relu(order_embedding[0])
    s2 = (V + 1.0) / (3.0 * (V - 1.0))
    scale = bn_weight * r * lax.rsqrt(r * r * s2 + 1e-5)
    shift = bn_bias
    nr_scale = float(2.0 / (V - 1.0))

    table_lin = class_embedding

    idxT = jnp.swapaxes(index_tensor, 0, 1)

    mesh = plsc.VectorSubcoreMesh(
        core_axis_name="c", subcore_axis_name="s",
        num_cores=_NC, num_subcores=_NS)

    run = pl.kernel(
        functools.partial(_body, L, B, nr_scale),
        out_type=jax.ShapeDtypeStruct((L, 4, _NW, 8, _BBLK), jnp.float32),
        mesh=mesh,
        scratch_types=[
            pltpu.VMEM((L, _BBLK), jnp.int32),
            pltpu.VMEM((_BURST * _BBLK, D), jnp.float32),
            pltpu.VMEM((_BURST, 4, 1, 8, _BBLK), jnp.float32),
            pltpu.VMEM((D,), jnp.float32),
            pltpu.VMEM((D,), jnp.float32),
            pltpu.SemaphoreType.DMA,
        ],
        compiler_params=pltpu.CompilerParams(
            use_tc_tiling_on_sc=False, needs_layout_passes=False),
    )
    out5 = run(table_lin, idxT, scale, shift)
    return out5.transpose(2, 4, 0, 1, 3).reshape(B, L, D)


# speedup vs baseline: 1.2216x; 1.2216x over previous
import functools

import jax
import jax.numpy as jnp
from jax import lax
from jax.experimental import pallas as pl
from jax.experimental.pallas import tpu as pltpu
from jax.experimental.pallas import tpu_sc as plsc

_NC = 2
_NS = 16
_NW = _NC * _NS
_LANES = 16
_BBLK = 128


_BURST = 8


def _body(L, B, nr_scale,
          table_hbm, idx_hbm, scale_hbm, shift_hbm, out_hbm,
          idx_v, rows_v, skew_v, rowsT_v, sc_v, sh_v, sg):
    wid = lax.axis_index("s") * _NC + lax.axis_index("c")
    b0 = pl.multiple_of(wid * _BBLK, _BBLK)
    pltpu.sync_copy(scale_hbm, sc_v)
    pltpu.sync_copy(shift_hbm, sh_v)
    a0 = sc_v[0:_LANES]
    a1 = sc_v[_LANES:2 * _LANES]
    c0 = sh_v[0:_LANES]
    c1 = sh_v[_LANES:2 * _LANES]
    pltpu.sync_copy(idx_hbm.at[:, pl.ds(b0, _BBLK)], idx_v)

    lane = lax.iota(jnp.int32, _LANES)
    t0 = lax.shift_right_logical(lane, 3)
    t1 = t0 + 2
    r8 = lane & 7
    zero = jnp.zeros((_LANES,), jnp.int32)

    @pl.loop(0, L // _BURST)
    def _burst(t):
        l0 = t * _BURST
        copies = []
        for j in range(_BURST):
            copies.append(pltpu.async_copy(
                table_hbm.at[idx_v.at[l0 + j]],
                rows_v.at[pl.ds(j * _BBLK, _BBLK)], sg))
        for c in copies:
            c.wait()

        @pl.loop(0, _BURST * _BBLK // _LANES)
        def _grp(gg):
            lg = lax.shift_right_logical(gg, 3)
            within = (gg & 7) * _LANES
            ivec = idx_v[l0 + lg, pl.ds(within, _LANES)]
            nrv = ivec.astype(jnp.float32) * nr_scale - 1.0
            for k in range(_LANES):
                nr = nrv[k]
                fr = gg * _LANES + k
                frv = zero + fr
                v0 = rows_v[fr, 0:_LANES]
                v1 = rows_v[fr, _LANES:2 * _LANES]
                plsc.store_scatter(skew_v, [frv, (lane + fr) & 31],
                                   v0 + (nr * a0 + c0))
                plsc.store_scatter(skew_v, [frv, (lane + (fr + _LANES)) & 31],
                                   v1 + (nr * a1 + c1))

        @pl.loop(0, _BURST)
        def _unskew(lg):
            rbase = lg * _BBLK
            for cc in range(_BBLK // _LANES):
                frv = lane + (rbase + cc * _LANES)
                for d in range(2 * _LANES):
                    v = plsc.load_gather(skew_v, [frv, (frv + d) & 31])
                    rowsT_v[lg, d // 8, 0, d % 8,
                            pl.ds(cc * _LANES, _LANES)] = v

        pltpu.sync_copy(rowsT_v,
                        out_hbm.at[pl.ds(l0, _BURST), :, pl.ds(wid, 1)])


def kernel(class_embedding, order_embedding, bn_weight, bn_bias, index_tensor):
    V, D = class_embedding.shape
    B, L = index_tensor.shape
    assert B == _NW * _BBLK and D == 2 * _LANES and L % _BURST == 0

    r = jax.nn.relu(order_embedding[0])
    s2 = (V + 1.0) / (3.0 * (V - 1.0))
    scale = bn_weight * r * lax.rsqrt(r * r * s2 + 1e-5)
    shift = bn_bias
    nr_scale = float(2.0 / (V - 1.0))

    table_lin = class_embedding

    idxT = jnp.swapaxes(index_tensor, 0, 1)

    mesh = plsc.VectorSubcoreMesh(
        core_axis_name="c", subcore_axis_name="s",
        num_cores=_NC, num_subcores=_NS)

    run = pl.kernel(
        functools.partial(_body, L, B, nr_scale),
        out_type=jax.ShapeDtypeStruct((L, 4, _NW, 8, _BBLK), jnp.float32),
        mesh=mesh,
        scratch_types=[
            pltpu.VMEM((L, _BBLK), jnp.int32),
            pltpu.VMEM((_BURST * _BBLK, D), jnp.float32),
            pltpu.VMEM((_BURST * _BBLK, D), jnp.float32),
            pltpu.VMEM((_BURST, 4, 1, 8, _BBLK), jnp.float32),
            pltpu.VMEM((D,), jnp.float32),
            pltpu.VMEM((D,), jnp.float32),
            pltpu.SemaphoreType.DMA,
        ],
        compiler_params=pltpu.CompilerParams(
            use_tc_tiling_on_sc=False, needs_layout_passes=False),
    )
    out5 = run(table_lin, idxT, scale, shift)
    return out5.transpose(2, 4, 0, 1, 3).reshape(B, L, D)


# speedup vs baseline: 1.3223x; 1.0824x over previous
import functools

import jax
import jax.numpy as jnp
from jax import lax
from jax.experimental import pallas as pl
from jax.experimental.pallas import tpu as pltpu
from jax.experimental.pallas import tpu_sc as plsc

_NC = 2
_NS = 16
_NW = _NC * _NS
_LANES = 16
_BBLK = 128


_BURST = 4


def _body(L, B, nr_scale,
          table_hbm, idx_hbm, scale_hbm, shift_hbm, out_hbm,
          idx_v, rows_v, skew_v, rowsT_v, sc_v, sh_v, sg, so):
    wid = lax.axis_index("s") * _NC + lax.axis_index("c")
    b0 = pl.multiple_of(wid * _BBLK, _BBLK)
    pltpu.sync_copy(scale_hbm, sc_v)
    pltpu.sync_copy(shift_hbm, sh_v)
    a0 = sc_v[0:_LANES]
    a1 = sc_v[_LANES:2 * _LANES]
    c0 = sh_v[0:_LANES]
    c1 = sh_v[_LANES:2 * _LANES]
    pltpu.sync_copy(idx_hbm.at[:, pl.ds(b0, _BBLK)], idx_v)

    lane = lax.iota(jnp.int32, _LANES)
    t0 = lax.shift_right_logical(lane, 3)
    t1 = t0 + 2
    r8 = lane & 7
    zero = jnp.zeros((_LANES,), jnp.int32)

    def fire_gathers(t, buf):
        handles = []
        for j in range(_BURST):
            handles.append(pltpu.async_copy(
                table_hbm.at[idx_v.at[t * _BURST + j]],
                rows_v.at[buf, pl.ds(j * _BBLK, _BBLK)], sg.at[buf]))
        return handles

    def drain_gathers(t, buf):
        for j in range(_BURST):
            pltpu.make_async_copy(
                table_hbm.at[idx_v.at[t * _BURST + j]],
                rows_v.at[buf, pl.ds(j * _BBLK, _BBLK)], sg.at[buf]).wait()

    def out_slice(l0):
        return out_hbm.at[pl.ds(l0, _BURST), :, pl.ds(wid, 1)]

    def process(t, buf):
        l0 = t * _BURST

        @pl.loop(0, _BURST * _BBLK // _LANES)
        def _grp(gg):
            lg = lax.shift_right_logical(gg, 3)
            within = (gg & 7) * _LANES
            ivec = idx_v[l0 + lg, pl.ds(within, _LANES)]
            nrv = ivec.astype(jnp.float32) * nr_scale - 1.0
            for k in range(_LANES):
                nr = nrv[k]
                fr = gg * _LANES + k
                frv = zero + fr
                v0 = rows_v[buf, fr, 0:_LANES]
                v1 = rows_v[buf, fr, _LANES:2 * _LANES]
                plsc.store_scatter(skew_v, [frv, (lane + fr) & 31],
                                   v0 + (nr * a0 + c0))
                plsc.store_scatter(skew_v, [frv, (lane + (fr + _LANES)) & 31],
                                   v1 + (nr * a1 + c1))

        @pl.when(t + 2 < L // _BURST)
        def _prefetch():
            fire_gathers(t + 2, buf)

        @pl.loop(0, _BURST)
        def _unskew(lg):
            rbase = lg * _BBLK
            for cc in range(_BBLK // _LANES):
                frv = lane + (rbase + cc * _LANES)
                for d in range(2 * _LANES):
                    v = plsc.load_gather(skew_v, [frv, (frv + d) & 31])
                    rowsT_v[buf, lg, d // 8, 0, d % 8,
                            pl.ds(cc * _LANES, _LANES)] = v

    fire_gathers(0, 0)
    fire_gathers(1, 1)

    @pl.loop(0, L // (2 * _BURST))
    def _pair(p):
        for buf in range(2):
            t = p * 2 + buf
            drain_gathers(t, buf)

            @pl.when(p > 0)
            def _wait_out():
                pltpu.make_async_copy(
                    rowsT_v.at[buf], out_slice(t * _BURST), so.at[buf]).wait()

            process(t, buf)
            pltpu.async_copy(rowsT_v.at[buf], out_slice(t * _BURST),
                             so.at[buf])

    for buf in range(2):
        pltpu.make_async_copy(
            rowsT_v.at[buf], out_slice(0), so.at[buf]).wait()


def kernel(class_embedding, order_embedding, bn_weight, bn_bias, index_tensor):
    V, D = class_embedding.shape
    B, L = index_tensor.shape
    assert B == _NW * _BBLK and D == 2 * _LANES and L % _BURST == 0

    r = jax.nn.relu(order_embedding[0])
    s2 = (V + 1.0) / (3.0 * (V - 1.0))
    scale = bn_weight * r * lax.rsqrt(r * r * s2 + 1e-5)
    shift = bn_bias
    nr_scale = float(2.0 / (V - 1.0))

    table_lin = class_embedding

    idxT = jnp.swapaxes(index_tensor, 0, 1)

    mesh = plsc.VectorSubcoreMesh(
        core_axis_name="c", subcore_axis_name="s",
        num_cores=_NC, num_subcores=_NS)

    run = pl.kernel(
        functools.partial(_body, L, B, nr_scale),
        out_type=jax.ShapeDtypeStruct((L, 4, _NW, 8, _BBLK), jnp.float32),
        mesh=mesh,
        scratch_types=[
            pltpu.VMEM((L, _BBLK), jnp.int32),
            pltpu.VMEM((2, _BURST * _BBLK, D), jnp.float32),
            pltpu.VMEM((_BURST * _BBLK, D), jnp.float32),
            pltpu.VMEM((2, _BURST, 4, 1, 8, _BBLK), jnp.float32),
            pltpu.VMEM((D,), jnp.float32),
            pltpu.VMEM((D,), jnp.float32),
            pltpu.SemaphoreType.DMA((2,)),
            pltpu.SemaphoreType.DMA((2,)),
        ],
        compiler_params=pltpu.CompilerParams(
            use_tc_tiling_on_sc=False, needs_layout_passes=False),
    )
    out5 = run(table_lin, idxT, scale, shift)
    return out5.transpose(2, 4, 0, 1, 3).reshape(B, L, D)


# speedup vs baseline: 1.3967x; 1.0563x over previous
import functools

import jax
import jax.numpy as jnp
from jax import lax
from jax.experimental import pallas as pl
from jax.experimental.pallas import tpu as pltpu
from jax.experimental.pallas import tpu_sc as plsc

_NC = 2
_NS = 16
_NW = _NC * _NS
_LANES = 16
_BBLK = 128


_BURST = 4


def _body(L, B, nr_scale,
          table_hbm, idx_hbm, scale_hbm, shift_hbm, out_hbm,
          idx_v, rows_v, skew_v, rowsT_v, sc_v, sh_v, sg, so):
    wid = lax.axis_index("s") * _NC + lax.axis_index("c")
    b0 = pl.multiple_of(wid * _BBLK, _BBLK)
    pltpu.sync_copy(scale_hbm, sc_v)
    pltpu.sync_copy(shift_hbm, sh_v)
    a0 = sc_v[0:_LANES]
    a1 = sc_v[_LANES:2 * _LANES]
    c0 = sh_v[0:_LANES]
    c1 = sh_v[_LANES:2 * _LANES]
    pltpu.sync_copy(idx_hbm.at[:, pl.ds(b0, _BBLK)], idx_v)

    lane = lax.iota(jnp.int32, _LANES)
    t0 = lax.shift_right_logical(lane, 3)
    t1 = t0 + 2
    r8 = lane & 7
    zero = jnp.zeros((_LANES,), jnp.int32)

    def fire_gathers(t, buf):
        handles = []
        for j in range(_BURST):
            handles.append(pltpu.async_copy(
                table_hbm.at[idx_v.at[t * _BURST + j]],
                rows_v.at[buf, pl.ds(j * _BBLK, _BBLK)], sg.at[buf]))
        return handles

    def drain_gathers(t, buf):
        for j in range(_BURST):
            pltpu.make_async_copy(
                table_hbm.at[idx_v.at[t * _BURST + j]],
                rows_v.at[buf, pl.ds(j * _BBLK, _BBLK)], sg.at[buf]).wait()

    def out_slice(l0):
        return out_hbm.at[pl.ds(l0, _BURST), :, pl.ds(wid, 1)]

    def process(t, buf):
        l0 = t * _BURST

        @plsc.parallel_loop(0, _BURST * _BBLK // _LANES)
        def _grp(gg):
            lg = lax.shift_right_logical(gg, 3)
            within = (gg & 7) * _LANES
            ivec = idx_v[l0 + lg, pl.ds(within, _LANES)]
            nrv = ivec.astype(jnp.float32) * nr_scale - 1.0
            for k in range(_LANES):
                nr = nrv[k]
                fr = gg * _LANES + k
                frv = zero + fr
                v0 = rows_v[buf, fr, 0:_LANES]
                v1 = rows_v[buf, fr, _LANES:2 * _LANES]
                plsc.store_scatter(skew_v, [frv, (lane + fr) & 31],
                                   v0 + (nr * a0 + c0))
                plsc.store_scatter(skew_v, [frv, (lane + (fr + _LANES)) & 31],
                                   v1 + (nr * a1 + c1))

        @pl.when(t + 2 < L // _BURST)
        def _prefetch():
            fire_gathers(t + 2, buf)

        @plsc.parallel_loop(0, _BURST)
        def _unskew(lg):
            rbase = lg * _BBLK
            for cc in range(_BBLK // _LANES):
                frv = lane + (rbase + cc * _LANES)
                for d in range(2 * _LANES):
                    v = plsc.load_gather(skew_v, [frv, (frv + d) & 31])
                    rowsT_v[buf, lg, d // 8, 0, d % 8,
                            pl.ds(cc * _LANES, _LANES)] = v

    fire_gathers(0, 0)
    fire_gathers(1, 1)

    @pl.loop(0, L // (2 * _BURST))
    def _pair(p):
        for buf in range(2):
            t = p * 2 + buf
            drain_gathers(t, buf)

            @pl.when(p > 0)
            def _wait_out():
                pltpu.make_async_copy(
                    rowsT_v.at[buf], out_slice(t * _BURST), so.at[buf]).wait()

            process(t, buf)
            pltpu.async_copy(rowsT_v.at[buf], out_slice(t * _BURST),
                             so.at[buf])

    for buf in range(2):
        pltpu.make_async_copy(
            rowsT_v.at[buf], out_slice(0), so.at[buf]).wait()


def kernel(class_embedding, order_embedding, bn_weight, bn_bias, index_tensor):
    V, D = class_embedding.shape
    B, L = index_tensor.shape
    assert B == _NW * _BBLK and D == 2 * _LANES and L % _BURST == 0

    r = jax.nn.relu(order_embedding[0])
    s2 = (V + 1.0) / (3.0 * (V - 1.0))
    scale = bn_weight * r * lax.rsqrt(r * r * s2 + 1e-5)
    shift = bn_bias
    nr_scale = float(2.0 / (V - 1.0))

    table_lin = class_embedding

    idxT = jnp.swapaxes(index_tensor, 0, 1)

    mesh = plsc.VectorSubcoreMesh(
        core_axis_name="c", subcore_axis_name="s",
        num_cores=_NC, num_subcores=_NS)

    run = pl.kernel(
        functools.partial(_body, L, B, nr_scale),
        out_type=jax.ShapeDtypeStruct((L, 4, _NW, 8, _BBLK), jnp.float32),
        mesh=mesh,
        scratch_types=[
            pltpu.VMEM((L, _BBLK), jnp.int32),
            pltpu.VMEM((2, _BURST * _BBLK, D), jnp.float32),
            pltpu.VMEM((_BURST * _BBLK, D), jnp.float32),
            pltpu.VMEM((2, _BURST, 4, 1, 8, _BBLK), jnp.float32),
            pltpu.VMEM((D,), jnp.float32),
            pltpu.VMEM((D,), jnp.float32),
            pltpu.SemaphoreType.DMA((2,)),
            pltpu.SemaphoreType.DMA((2,)),
        ],
        compiler_params=pltpu.CompilerParams(
            use_tc_tiling_on_sc=False, needs_layout_passes=False),
    )
    out5 = run(table_lin, idxT, scale, shift)
    return out5.transpose(2, 4, 0, 1, 3).reshape(B, L, D)
